# baseline (speedup 1.0000x reference)
import functools

import jax
import jax.numpy as jnp
from jax.experimental import pallas as pl
import jax.experimental.pallas.tpu as pltpu

N = 10000
BR = 400
NB = N // BR


def _gcn_kernel(x_ref, adj_ref, W1_ref, b1_ref, W2_ref, b2_ref, Wl_ref, bl_ref,
                logp_ref, z_ref, s1_ref, h_ref):
    phase = pl.program_id(0)
    i = pl.program_id(1)

    @pl.when(jnp.logical_and(phase == 0, i == 0))
    def _():
        s1_ref[:] = jnp.dot(x_ref[:], W1_ref[:],
                            preferred_element_type=jnp.float32)

    @pl.when(phase == 0)
    def _():
        hb = jnp.dot(adj_ref[:], s1_ref[:],
                     preferred_element_type=jnp.float32) + b1_ref[:]
        h_ref[pl.ds(i * BR, BR), :] = jnp.maximum(hb, 0.0)

    @pl.when(phase == 1)
    def _():
        t = jnp.dot(adj_ref[:], h_ref[:], preferred_element_type=jnp.float32)
        zb = jnp.dot(t, W2_ref[:], preferred_element_type=jnp.float32) + b2_ref[:]
        z_ref[:] = zb
        logits = jnp.dot(jnp.maximum(zb, 0.0), Wl_ref[:],
                         preferred_element_type=jnp.float32) + bl_ref[:]
        m = jnp.max(logits, axis=1, keepdims=True)
        s = logits - m
        lse = jnp.log(jnp.sum(jnp.exp(s), axis=1, keepdims=True))
        logp_ref[:] = s - lse


@jax.jit
def kernel(x, adj, W1, b1, W2, b2, Wl, bl):
    in_feat = x.shape[1]
    n_hid = W1.shape[1]
    n_emb = W2.shape[1]
    n_class = Wl.shape[1]

    b1r = b1.reshape(1, n_hid)
    b2r = b2.reshape(1, n_emb)
    blr = bl.reshape(1, n_class)

    full = lambda p, i: (0, 0)
    out_idx = lambda p, i: (jnp.where(p == 1, i, 0), 0)

    logp, z = pl.pallas_call(
        _gcn_kernel,
        grid=(2, NB),
        in_specs=[
            pl.BlockSpec((N, in_feat), full),
            pl.BlockSpec((BR, N), lambda p, i: (i, 0)),
            pl.BlockSpec((in_feat, n_hid), full),
            pl.BlockSpec((1, n_hid), full),
            pl.BlockSpec((n_hid, n_emb), full),
            pl.BlockSpec((1, n_emb), full),
            pl.BlockSpec((n_emb, n_class), full),
            pl.BlockSpec((1, n_class), full),
        ],
        out_specs=[
            pl.BlockSpec((BR, n_class), out_idx),
            pl.BlockSpec((BR, n_emb), out_idx),
        ],
        out_shape=[
            jax.ShapeDtypeStruct((N, n_class), jnp.float32),
            jax.ShapeDtypeStruct((N, n_emb), jnp.float32),
        ],
        scratch_shapes=[
            pltpu.VMEM((N, n_hid), jnp.float32),
            pltpu.VMEM((N, n_hid), jnp.float32),
        ],
    )(x, adj, W1, b1r, W2, b2r, Wl, blr)
    return (logp, z)
